# baseline (speedup 1.0000x reference)
import functools

import jax
import jax.numpy as jnp
from jax import lax
from jax.experimental import pallas as pl
from jax.experimental.pallas import tpu as pltpu
from jax.experimental.pallas import tpu_sc as plsc

N = 10000
E = 320000
G = 100
K = 30

_NPr = 10112
_TPT = _NPr // 16
_NCHUNK = 79
_EPAD = 32 * _NCHUNK * 128
_NPAD2 = 10240

@functools.lru_cache(maxsize=None)
def _make_edge_agg(W):
    @functools.partial(
        pl.kernel,
        out_type=jax.ShapeDtypeStruct((2, _NPr, W), jnp.float32),
        mesh=plsc.VectorSubcoreMesh(core_axis_name="c", subcore_axis_name="s"),
        compiler_params=pltpu.CompilerParams(use_tc_tiling_on_sc=False),
        scratch_types=[
            pltpu.VMEM((_NCHUNK, 128), jnp.int32),
            pltpu.VMEM((_NCHUNK, 128), jnp.int32),
            pltpu.VMEM((128, W), jnp.float32),
            pltpu.VMEM((_TPT, W), jnp.float32),
            pltpu.VMEM_SHARED((_NPr, W), jnp.float32),
            pltpu.SemaphoreType.DMA,
        ],
    )
    def k(table_hbm, srci_hbm, dsti_hbm, zeros_hbm, out_hbm,
          sidx, didx, rows, tbuf, acc, sem):
        c = lax.axis_index("c")
        s = lax.axis_index("s")
        wid = c * 16 + s
        pltpu.sync_copy(srci_hbm.at[wid], sidx)
        pltpu.sync_copy(dsti_hbm.at[wid], didx)
        pltpu.sync_copy(zeros_hbm, tbuf)
        pltpu.sync_copy(tbuf, acc.at[pl.ds(s * _TPT, _TPT)])
        plsc.subcore_barrier()

        def body(j, carry):
            pltpu.async_copy(table_hbm.at[sidx.at[j]], rows, sem).wait()
            pltpu.sync_copy(rows, acc.at[didx.at[j]], add=True)
            return carry

        lax.fori_loop(0, _NCHUNK, body, 0)
        plsc.subcore_barrier()
        pltpu.sync_copy(acc.at[pl.ds(s * _TPT, _TPT)], tbuf)
        pltpu.sync_copy(tbuf, out_hbm.at[c, pl.ds(s * _TPT, _TPT)])

    return k


def _sc_agg32(*args):
    return _make_edge_agg(32)(*args)


def _sc_agg16(*args):
    return _make_edge_agg(16)(*args)


@functools.lru_cache(maxsize=None)
def _make_rowgather():
    @functools.partial(
        pl.kernel,
        out_type=jax.ShapeDtypeStruct((3072, 128), jnp.float32),
        mesh=plsc.VectorSubcoreMesh(core_axis_name="c", subcore_axis_name="s"),
        compiler_params=pltpu.CompilerParams(use_tc_tiling_on_sc=False),
        scratch_types=[
            pltpu.VMEM((96,), jnp.int32),
            pltpu.VMEM((96, 128), jnp.float32),
            pltpu.SemaphoreType.DMA,
        ],
    )
    def k(table_hbm, idx_hbm, out_hbm, idxv, rows, sem):
        c = lax.axis_index("c")
        s = lax.axis_index("s")
        wid = c * 16 + s
        pltpu.sync_copy(idx_hbm.at[wid], idxv)
        pltpu.async_copy(table_hbm.at[idxv], rows, sem).wait()
        pltpu.sync_copy(rows, out_hbm.at[pl.ds(wid * 96, 96)])

    return k


def _sc_rowgather(*args):
    return _make_rowgather()(*args)


def _prep_body(s_ref, d_ref, o_ref):
    s = s_ref[...]
    d = d_ref[...]
    e = (lax.broadcasted_iota(jnp.int32, s.shape, 0) * 128
         + lax.broadcasted_iota(jnp.int32, s.shape, 1))
    o_ref[...] = jnp.where(s == d, N + (e & 15), d)


def _layer0_body(x_ref, w_ref, degp_ref, hs_ref, dis_ref):
    degp = degp_ref[...]
    deg = degp[0, :N, 0:1] + degp[1, :N, 0:1]
    dis = lax.rsqrt(deg + 1.0)
    h = jnp.dot(x_ref[...], w_ref[...], preferred_element_type=jnp.float32)
    hs_ref[...] = dis * h
    dis_ref[...] = dis


def _combine_body(p_ref, hs_ref, dis_ref, b_ref, wn_ref, x_ref, hsn_ref):
    p = p_ref[...]
    s = p[0, :N, :] + p[1, :N, :]
    dis = dis_ref[...]
    xl = jnp.tanh(dis * (s + hs_ref[...]) + b_ref[...])
    x_ref[...] = xl
    hsn_ref[...] = dis * jnp.dot(xl, wn_ref[...],
                                 preferred_element_type=jnp.float32)


def _last_body(p_ref, hs_ref, dis_ref, b_ref, v_ref):
    p = p_ref[...]
    s = p[0, :N, :] + p[1, :N, :]
    t = jnp.tanh(dis_ref[...] * (s + hs_ref[...]) + b_ref[...])
    v_ref[...] = t[:, 0:1]


def _topk_body(v_ref, b_ref, o_ref):
    g = pl.program_id(0)
    v = v_ref[...]
    b = b_ref[...]
    neg = jnp.float32(-jnp.inf)
    work = jnp.where(b == g, v, neg)
    flat = (lax.broadcasted_iota(jnp.int32, work.shape, 0) * 128
            + lax.broadcasted_iota(jnp.int32, work.shape, 1))
    lane = lax.broadcasted_iota(jnp.int32, (1, 32), 1)
    out = jnp.zeros((1, 32), jnp.int32)
    for j in range(K):
        m = jnp.max(work)
        valid = m > neg
        cand = jnp.where(work == m, flat, jnp.int32(2 ** 30))
        idx = jnp.min(cand)
        sel = jnp.where(valid, idx, N + ((g * K + j) & 7))
        out = jnp.where(lane == j, sel, out)
        work = jnp.where(flat == idx, neg, work)
    o_ref[...] = out.reshape(1, 1, 32)


def _head1_body(g_ref, w_ref, b_ref, o_ref):
    o_ref[...] = jax.nn.relu(
        jnp.dot(g_ref[...], w_ref[...], preferred_element_type=jnp.float32)
        + b_ref[...])


def _head2_body(a_ref, b2_ref, w_ref, bb_ref, o_ref):
    z = jnp.maximum(a_ref[...], b2_ref[...])
    o_ref[...] = jax.nn.relu(
        jnp.dot(z, w_ref[...], preferred_element_type=jnp.float32)
        + bb_ref[...])


def _head3_body(y_ref, fw_ref, fb_ref, gw_ref, gb_ref, o_ref):
    h = jax.nn.relu(
        jnp.dot(y_ref[...], fw_ref[...], preferred_element_type=jnp.float32)
        + fb_ref[...])
    logits = jnp.dot(h, gw_ref[...],
                     preferred_element_type=jnp.float32) + gb_ref[...]
    m = jnp.max(logits, axis=1, keepdims=True)
    lse = m + jnp.log(jnp.sum(jnp.exp(logits - m), axis=1, keepdims=True))
    o_ref[...] = logits - lse


def _sds(shape):
    return jax.ShapeDtypeStruct(shape, jnp.float32)


def kernel(x, edge_index, batch, W1, b1, W2, b2, W3, b3, W4, b4, conv5_w,
           conv5_b, conv6_w, conv6_b, fc1_w, fc1_b, fc2_w, fc2_b):
    f32 = jnp.float32
    i32 = jnp.int32
    src = edge_index[0]
    dst = edge_index[1]

    dste2d = pl.pallas_call(
        _prep_body,
        out_shape=jax.ShapeDtypeStruct((2500, 128), i32),
    )(src.reshape(2500, 128), dst.reshape(2500, 128))
    npad = _EPAD - E
    pad_dst = N + (jnp.arange(npad, dtype=i32) & 15)
    pad_src = (jnp.arange(npad, dtype=i32) * 97) % N
    dsti = jnp.concatenate([dste2d.reshape(E), pad_dst]).reshape(32, _NCHUNK, 128)
    srci = jnp.concatenate([src, pad_src]).reshape(32, _NCHUNK, 128)
    z32 = jnp.zeros((_TPT, 32), f32)
    z16 = jnp.zeros((_TPT, 16), f32)

    ones_t = jnp.concatenate([jnp.ones((N, 1), f32), jnp.zeros((N, 15), f32)], 1)
    degp = _sc_agg16(ones_t, srci, dsti, z16)

    hs1, dis = pl.pallas_call(
        _layer0_body,
        out_shape=[_sds((N, 32)), _sds((N, 1))],
    )(x, W1, degp)

    def combine(P, hs, b2d, Wn, wout):
        return pl.pallas_call(
            _combine_body,
            out_shape=[_sds((N, 32)), _sds((N, wout))],
        )(P, hs, dis, b2d, Wn)

    P1 = _sc_agg32(hs1, srci, dsti, z32)
    x1, hs2 = combine(P1, hs1, b1.reshape(1, 32), W2, 32)
    P2 = _sc_agg32(hs2, srci, dsti, z32)
    x2, hs3 = combine(P2, hs2, b2.reshape(1, 32), W3, 32)
    P3 = _sc_agg32(hs3, srci, dsti, z32)
    W4p = jnp.pad(W4, ((0, 0), (0, 15)))
    x3, hs4 = combine(P3, hs3, b3.reshape(1, 32), W4p, 16)
    P4 = _sc_agg16(hs4, srci, dsti, z16)
    b4p = jnp.pad(b4.reshape(1, 1), ((0, 0), (0, 15)))
    vcol = pl.pallas_call(
        _last_body,
        out_shape=_sds((N, 1)),
    )(P4, hs4, dis, b4p)

    v2d = jnp.pad(vcol.reshape(N), (0, _NPAD2 - N),
                  constant_values=-jnp.inf).reshape(80, 128)
    batch2d = jnp.pad(batch, (0, _NPAD2 - N),
                      constant_values=-1).reshape(80, 128)
    idx3 = pl.pallas_call(
        _topk_body,
        grid=(G,),
        in_specs=[
            pl.BlockSpec((80, 128), lambda g: (0, 0)),
            pl.BlockSpec((80, 128), lambda g: (0, 0)),
        ],
        out_specs=pl.BlockSpec((1, 1, 32), lambda g: (g, 0, 0)),
        out_shape=jax.ShapeDtypeStruct((G, 1, 32), i32),
    )(v2d, batch2d)
    idxs = idx3[:, 0, :K].reshape(G * K)
    pad_g = N + (jnp.arange(72, dtype=i32) & 7)
    idx_g = jnp.concatenate([idxs, pad_g]).reshape(32, 96)

    xc = jnp.concatenate([x1, x2, x3, vcol, jnp.zeros((N, 31), f32)], 1)
    xc_ext = jnp.pad(xc, ((0, 16), (0, 0)))
    rows = _sc_rowgather(xc_ext, idx_g)

    W5p = jnp.pad(conv5_w[:, 0, :].T, ((0, 31), (0, 0)))
    Y5 = pl.pallas_call(
        _head1_body,
        out_shape=_sds((3072, 16)),
    )(rows, W5p, conv5_b.reshape(1, 16))

    Y5r = Y5[:G * K].reshape(G, K, 16)
    Y5e = Y5r[:, 0::2, :]
    Y5o = Y5r[:, 1::2, :]
    A2 = jnp.stack([Y5e[:, t:t + 5, :] for t in range(11)], 1).reshape(1100, 80)
    B2 = jnp.stack([Y5o[:, t:t + 5, :] for t in range(11)], 1).reshape(1100, 80)
    W6m = conv6_w.transpose(2, 1, 0).reshape(80, 32)
    Y6 = pl.pallas_call(
        _head2_body,
        out_shape=_sds((1100, 32)),
    )(A2, B2, W6m, conv6_b.reshape(1, 32))

    fc1_wr = fc1_w.reshape(32, 11, 128).transpose(1, 0, 2).reshape(352, 128)
    out = pl.pallas_call(
        _head3_body,
        out_shape=_sds((G, 10)),
    )(Y6.reshape(G, 352), fc1_wr, fc1_b.reshape(1, 128),
      fc2_w, fc2_b.reshape(1, 10))
    return out
